# baseline (speedup 1.0000x reference)
import functools
import jax
import jax.numpy as jnp
from jax import lax
from jax.experimental import pallas as pl
from jax.experimental.pallas import tpu as pltpu
from jax.experimental.pallas import tpu_sc as plsc

_B, _S, _D = 4, 8192, 1024
_NW = 32
_ROWS_PER_W = _S // _NW
_C = 4
_NCHUNK = _ROWS_PER_W // _C
_NBUF = 4
_LANES = 16
_GPR = _D // _LANES


def _sc_body(x_hbm, emb_hbm, out_hbm, emb_v, x_v,
             in_s0, in_s1, in_s2, in_s3, out_s0, out_s1, out_s2, out_s3):
    in_sems = (in_s0, in_s1, in_s2, in_s3)
    out_sems = (out_s0, out_s1, out_s2, out_s3)
    wid = lax.axis_index("s") * 2 + lax.axis_index("c")
    row0 = wid * _ROWS_PER_W

    def start_in(c, u):
        r = row0 + c * _C
        pltpu.make_async_copy(
            emb_hbm.at[pl.ds(r, _C)], emb_v.at[u], in_sems[u]).start()
        pltpu.make_async_copy(
            x_hbm.at[:, pl.ds(r, _C)], x_v.at[u], in_sems[u]).start()

    def wait_in(c, u):
        r = row0 + c * _C
        pltpu.make_async_copy(
            emb_hbm.at[pl.ds(r, _C)], emb_v.at[u], in_sems[u]).wait()
        pltpu.make_async_copy(
            x_hbm.at[:, pl.ds(r, _C)], x_v.at[u], in_sems[u]).wait()

    def start_out(c, u):
        r = row0 + c * _C
        pltpu.make_async_copy(
            x_v.at[u], out_hbm.at[:, pl.ds(r, _C)], out_sems[u]).start()

    def wait_out(c, u):
        r = row0 + c * _C
        pltpu.make_async_copy(
            x_v.at[u], out_hbm.at[:, pl.ds(r, _C)], out_sems[u]).wait()

    def compute(u):
        for row in range(_C):
            def col_body(k, c2, row=row):
                for v in range(4):
                    col = (k * 4 + v) * _LANES
                    e = emb_v[u, row, pl.ds(col, _LANES)]
                    for b in range(_B):
                        plsc.addupdate(
                            x_v.at[u, b, row, pl.ds(col, _LANES)], e)
                return c2
            lax.fori_loop(0, _GPR // 4, col_body, 0)

    start_in(0, 0)
    start_in(1, 1)

    def outer(i, carry):
        c0 = i * _NBUF
        for u in range(_NBUF):
            c = c0 + u
            uo = (u + 2) % _NBUF

            @pl.when(c >= 2)
            def _():
                wait_out(c - 2, uo)

            @pl.when(c + 2 < _NCHUNK)
            def _():
                start_in(c + 2, uo)

            wait_in(c, u)
            compute(u)
            start_out(c, u)
        return carry

    lax.fori_loop(0, _NCHUNK // _NBUF, outer, 0)
    wait_out(_NCHUNK - 2, (_NCHUNK - 2) % _NBUF)
    wait_out(_NCHUNK - 1, (_NCHUNK - 1) % _NBUF)


def kernel(x, embeddings):
    mesh = plsc.VectorSubcoreMesh(core_axis_name="c", subcore_axis_name="s")
    run = functools.partial(
        pl.kernel,
        mesh=mesh,
        out_type=jax.ShapeDtypeStruct((_B, _S, _D), jnp.float32),
        scratch_types=[
            pltpu.VMEM((_NBUF, _C, _D), jnp.float32),
            pltpu.VMEM((_NBUF, _B, _C, _D), jnp.float32),
        ] + [pltpu.SemaphoreType.DMA] * (2 * _NBUF),
    )(_sc_body)
    return run(x, embeddings)
